# baseline (speedup 1.0000x reference)
import dataclasses
import functools

import jax
import jax.numpy as jnp
from jax import lax
from jax.experimental import pallas as pl
from jax.experimental.pallas import tpu as pltpu
from jax.experimental.pallas import tpu_sc as plsc

NC = 2
NS = 16
NW = NC * NS
L = 16

N_STRUCT = 512
NB = 4


def _sc_partials(e, z, b, s1, s2, chunk):
    mesh = plsc.VectorSubcoreMesh(core_axis_name="c", subcore_axis_name="s")
    tbl = s1.shape[0]
    n = e.shape[0]
    blk = chunk // NB
    cp = pltpu.CompilerParams()
    if "needs_layout_passes" in pltpu.CompilerParams.__dataclass_fields__:
        cp = dataclasses.replace(cp, needs_layout_passes=False)

    @functools.partial(
        pl.kernel,
        out_type=jax.ShapeDtypeStruct((NW, N_STRUCT), jnp.float32),
        mesh=mesh,
        compiler_params=cp,
        scratch_types=[
            pltpu.VMEM((blk,), jnp.float32),
            pltpu.VMEM((blk,), jnp.float32),
            pltpu.VMEM((blk,), jnp.int32),
            pltpu.VMEM((blk,), jnp.int32),
            pltpu.VMEM((blk,), jnp.int32),
            pltpu.VMEM((blk,), jnp.int32),
            pltpu.VMEM((tbl,), jnp.float32),
            pltpu.VMEM((tbl,), jnp.float32),
            pltpu.VMEM((L * N_STRUCT,), jnp.float32),
            pltpu.VMEM((N_STRUCT,), jnp.float32),
            pltpu.SemaphoreType.DMA,
            pltpu.SemaphoreType.DMA,
            pltpu.SemaphoreType.DMA,
            pltpu.SemaphoreType.DMA,
            pltpu.SemaphoreType.DMA,
            pltpu.SemaphoreType.DMA,
            pltpu.SemaphoreType.DMA,
            pltpu.SemaphoreType.DMA,
        ],
    )
    def k(e_hbm, z_hbm, b_hbm, s1_hbm, s2_hbm, out_hbm,
          e_v0, e_v1, z_v0, z_v1, b_v0, b_v1, s1_v, s2_v, acc_v, red_v,
          se0, se1, sz0, sz1, sb0, sb1, ss1, ss2):
        bufs = [(e_v0, z_v0, b_v0), (e_v1, z_v1, b_v1)]
        wid = lax.axis_index("c") * NS + lax.axis_index("s")
        nominal = wid * chunk
        base = jnp.minimum(nominal, n - chunk)
        start = jnp.minimum(nominal - base, chunk)
        sems = [(se0, sz0, sb0), (se1, sz1, sb1)]

        def issue(blki, par):
            off = base + blki * blk
            ce = pltpu.async_copy(e_hbm.at[pl.ds(off, blk)], bufs[par][0], sems[par][0])
            cz = pltpu.async_copy(z_hbm.at[pl.ds(off, blk)], bufs[par][1], sems[par][1])
            cb = pltpu.async_copy(b_hbm.at[pl.ds(off, blk)], bufs[par][2], sems[par][2])
            return (ce, cz, cb)

        cps = issue(0, 0)
        cs1 = pltpu.async_copy(s1_hbm, s1_v, ss1)
        cs2 = pltpu.async_copy(s2_hbm, s2_v, ss2)

        zeros = jnp.zeros((L,), jnp.float32)

        @plsc.parallel_loop(0, L * N_STRUCT, step=L, unroll=4)
        def _(i):
            acc_v[pl.ds(i, L)] = zeros

        @plsc.parallel_loop(0, N_STRUCT, step=L, unroll=4)
        def _(i):
            red_v[pl.ds(i, L)] = zeros

        lane_base = lax.iota(jnp.int32, L) * N_STRUCT
        cs1.wait()
        cs2.wait()

        def run_block(par, lo):
            ev, zv, bv = bufs[par]

            def atoms(i):
                zz = zv[pl.ds(i, L)]
                ee = ev[pl.ds(i, L)]
                ss = plsc.load_gather(s1_v, [zz]) * plsc.load_gather(s2_v, [zz])
                bb = bv[pl.ds(i, L)]
                plsc.addupdate_scatter(acc_v, [lane_base + bb], ee * ss)

            head = lo + (blk - lo) % (4 * L)

            @pl.loop(lo, head, step=L)
            def _(i):
                atoms(i)

            @plsc.parallel_loop(head, blk, step=L, unroll=4)
            def _(i):
                atoms(i)

        blo = None
        for blki in range(NB):
            par = blki % 2
            if blki + 1 < NB:
                nxt = issue(blki + 1, 1 - par)
            for c in cps:
                c.wait()
            if blki == 0:
                blo = jnp.min(b_v0[pl.ds(jnp.clip(start, 0, blk - L), L)])
            run_block(par, jnp.clip(start - blki * blk, 0, blk))
            if blki + 1 < NB:
                cps = nxt

        bhi = jnp.max(bufs[(NB - 1) % 2][2][pl.ds(blk - L, L)])
        clo = (blo // L) * L
        chi = (bhi // L) * L + L

        @plsc.parallel_loop(clo, chi, step=L)
        def _(c):
            t = [acc_v[pl.ds(r * N_STRUCT + c, L)] for r in range(L)]
            while len(t) > 1:
                t = [x + y for x, y in zip(t[::2], t[1::2])]
            red_v[pl.ds(c, L)] = t[0]

        pltpu.sync_copy(red_v, out_hbm.at[wid])

    return k(e, z, b, s1, s2)


def _tc_reduce(partials):
    def body(x_ref, o_ref):
        o_ref[...] = jnp.sum(x_ref[...], axis=0)

    return pl.pallas_call(
        body,
        out_shape=jax.ShapeDtypeStruct((N_STRUCT,), jnp.float32),
    )(partials)


def kernel(local_energies, scales1, scales2, Z, batch):
    e = local_energies
    z = Z.astype(jnp.int32)
    b = batch.astype(jnp.int32)
    n = e.shape[0]
    if n % L:
        pad = L - n % L
        e = jnp.concatenate([e, jnp.zeros((pad,), jnp.float32)])
        z = jnp.concatenate([z, jnp.zeros((pad,), jnp.int32)])
        b = jnp.concatenate([b, jnp.zeros((pad,), jnp.int32)])
        n += pad
    chunk = -(-n // (NW * L * NB)) * (L * NB)
    if chunk > n:
        chunk = -(-n // (L * NB)) * (L * NB)
        pad = chunk - n
        if pad:
            e = jnp.concatenate([e, jnp.zeros((pad,), jnp.float32)])
            z = jnp.concatenate([z, jnp.zeros((pad,), jnp.int32)])
            b = jnp.concatenate([b, jnp.zeros((pad,), jnp.int32)])
            n = chunk
    partials = _sc_partials(e, z, b, scales1, scales2, chunk)
    return _tc_reduce(partials)


# speedup vs baseline: 1.0939x; 1.0939x over previous
import dataclasses
import functools

import jax
import jax.numpy as jnp
from jax import lax
from jax.experimental import pallas as pl
from jax.experimental.pallas import tpu as pltpu
from jax.experimental.pallas import tpu_sc as plsc

NC = 2
NS = 16
NW = NC * NS
L = 16

N_STRUCT = 512


def _sc_partials(e, z, b, s1, s2, chunk):
    mesh = plsc.VectorSubcoreMesh(core_axis_name="c", subcore_axis_name="s")
    tbl = s1.shape[0]
    n = e.shape[0]
    cp = pltpu.CompilerParams()
    if "needs_layout_passes" in pltpu.CompilerParams.__dataclass_fields__:
        cp = dataclasses.replace(cp, needs_layout_passes=False)

    @functools.partial(
        pl.kernel,
        out_type=jax.ShapeDtypeStruct((NW, N_STRUCT), jnp.float32),
        mesh=mesh,
        compiler_params=cp,
        scratch_types=[
            pltpu.VMEM((chunk,), jnp.float32),
            pltpu.VMEM((chunk,), jnp.int32),
            pltpu.VMEM((chunk,), jnp.int32),
            pltpu.VMEM((tbl,), jnp.float32),
            pltpu.VMEM((tbl,), jnp.float32),
            pltpu.VMEM((L * (N_STRUCT + 1),), jnp.float32),
            pltpu.VMEM((N_STRUCT,), jnp.float32),
            pltpu.SemaphoreType.DMA,
            pltpu.SemaphoreType.DMA,
            pltpu.SemaphoreType.DMA,
            pltpu.SemaphoreType.DMA,
            pltpu.SemaphoreType.DMA,
        ],
    )
    def k(e_hbm, z_hbm, b_hbm, s1_hbm, s2_hbm, out_hbm,
          e_v, z_v, b_v, s1_v, s2_v, acc_v, red_v,
          sem_e, sem_z, sem_b, sem_s1, sem_s2):
        wid = lax.axis_index("c") * NS + lax.axis_index("s")
        nominal = wid * chunk
        base = jnp.minimum(nominal, n - chunk)
        start = jnp.minimum(nominal - base, chunk)
        cp_e = pltpu.async_copy(e_hbm.at[pl.ds(base, chunk)], e_v, sem_e)
        cp_z = pltpu.async_copy(z_hbm.at[pl.ds(base, chunk)], z_v, sem_z)
        cp_b = pltpu.async_copy(b_hbm.at[pl.ds(base, chunk)], b_v, sem_b)
        cp_s1 = pltpu.async_copy(s1_hbm, s1_v, sem_s1)
        cp_s2 = pltpu.async_copy(s2_hbm, s2_v, sem_s2)

        zeros = jnp.zeros((L,), jnp.float32)

        @plsc.parallel_loop(0, N_STRUCT, step=L, unroll=4)
        def _(i):
            red_v[pl.ds(i, L)] = zeros

        cp_b.wait()
        blo = jnp.min(b_v[pl.ds(start, L)])
        bhi = jnp.max(b_v[pl.ds(chunk - L, L)])
        clo = (blo // L) * L
        chi = (bhi // L) * L + L

        stride = N_STRUCT + 1
        iota = lax.iota(jnp.int32, L)
        lane_base = iota * stride

        @plsc.parallel_loop(clo, chi, step=L)
        def _(c):
            for r in range(L):
                plsc.store_scatter(acc_v, [r * stride + c + iota], zeros)
        cp_e.wait()
        cp_z.wait()
        cp_s1.wait()
        cp_s2.wait()

        def atoms(i):
            zz = z_v[pl.ds(i, L)]
            ee = e_v[pl.ds(i, L)]
            ss = plsc.load_gather(s1_v, [zz]) * plsc.load_gather(s2_v, [zz])
            bb = b_v[pl.ds(i, L)]
            plsc.addupdate_scatter(acc_v, [lane_base + bb], ee * ss)

        head = start + (chunk - start) % (4 * L)

        @pl.loop(start, head, step=L)
        def _(i):
            atoms(i)

        @plsc.parallel_loop(head, chunk, step=L, unroll=4)
        def _(i):
            atoms(i)

        @plsc.parallel_loop(clo, chi, step=L)
        def _(c):
            t = [plsc.load_gather(acc_v, [r * stride + c + iota])
                 for r in range(L)]
            while len(t) > 1:
                t = [a + b for a, b in zip(t[::2], t[1::2])]
            red_v[pl.ds(c, L)] = t[0]

        pltpu.sync_copy(red_v, out_hbm.at[wid])

    return k(e, z, b, s1, s2)


def _tc_reduce(partials):
    def body(x_ref, o_ref):
        o_ref[...] = jnp.sum(x_ref[...], axis=0)

    return pl.pallas_call(
        body,
        out_shape=jax.ShapeDtypeStruct((N_STRUCT,), jnp.float32),
    )(partials)


def kernel(local_energies, scales1, scales2, Z, batch):
    e = local_energies
    z = Z.astype(jnp.int32)
    b = batch.astype(jnp.int32)
    n = e.shape[0]
    if n % L:
        pad = L - n % L
        e = jnp.concatenate([e, jnp.zeros((pad,), jnp.float32)])
        z = jnp.concatenate([z, jnp.zeros((pad,), jnp.int32)])
        b = jnp.concatenate([b, jnp.zeros((pad,), jnp.int32)])
        n += pad
    chunk = -(-n // (NW * L)) * L
    if chunk > n:
        chunk = n
    partials = _sc_partials(e, z, b, scales1, scales2, chunk)
    return _tc_reduce(partials)
